# Initial kernel scaffold; baseline (speedup 1.0000x reference)
import jax
import jax.numpy as jnp
from jax.experimental import pallas as pl


def kernel(x, pe_table):
    raise NotImplementedError("write your pallas kernel here")


# speedup vs baseline: 1.9310x; 1.9310x over previous
import jax
import jax.numpy as jnp
from jax.experimental import pallas as pl


def _add_body(x_ref, pe_ref, out_ref):
    out_ref[...] = x_ref[...] + pe_ref[...][None, :, :]


def kernel(x, pe_table):
    B, L, D = x.shape
    BL = 512
    grid = (L // BL, B)
    return pl.pallas_call(
        _add_body,
        grid=grid,
        in_specs=[
            pl.BlockSpec((1, BL, D), lambda i, b: (b, i, 0)),
            pl.BlockSpec((BL, D), lambda i, b: (i, 0)),
        ],
        out_specs=pl.BlockSpec((1, BL, D), lambda i, b: (b, i, 0)),
        out_shape=jax.ShapeDtypeStruct((B, L, D), x.dtype),
    )(x, pe_table)
